# baseline (speedup 1.0000x reference)
import jax
import jax.numpy as jnp
from jax import lax
from jax.experimental import pallas as pl
from jax.experimental.pallas import tpu as pltpu
from jax.experimental.pallas import tpu_sc as plsc

D_MODEL = 2048
SEQ = 8192
MAXC = 16
LANES = 16
NC = 2
NS = 16
NW = NC * NS
ROWS_W = SEQ // NW
K = 4
KT = 1


def _sc_body(ctab_hbm, srows_hbm, out_hbm, row_v, srow_v, tile_v, shared, sem):
    cid = lax.axis_index("c")
    sid = lax.axis_index("s")
    wid = sid * NC + cid
    base = wid * ROWS_W
    pltpu.sync_copy(ctab_hbm.at[pl.ds(sid, 1)], row_v)
    pltpu.sync_copy(srows_hbm.at[pl.ds(sid, 1)], srow_v)

    def add_chunk(t, carry):
        sl = pl.ds(t * LANES, LANES)
        row_v[0, sl] = row_v[0, sl] + srow_v[0, sl]
        return carry

    lax.fori_loop(0, D_MODEL // LANES, add_chunk, 0)
    for k in range(K):
        pltpu.sync_copy(row_v, shared.at[pl.ds(k * MAXC + sid, 1)])
    plsc.subcore_barrier()
    pltpu.sync_copy(shared.at[pl.ds(0, KT * MAXC)], tile_v)
    sp_rows = (K + 3) * MAXC
    copies = [
        pltpu.async_copy(
            tile_v,
            out_hbm.at[pl.ds(base + sp_rows + t * (KT * MAXC), KT * MAXC)],
            sem,
        )
        for t in range(9)
    ]
    copies += [
        pltpu.async_copy(shared, out_hbm.at[pl.ds(base, K * MAXC)], sem),
        pltpu.async_copy(
            shared.at[pl.ds(0, 2 * MAXC)],
            out_hbm.at[pl.ds(base + K * MAXC, 2 * MAXC)],
            sem,
        ),
        pltpu.async_copy(
            shared.at[pl.ds(0, MAXC)],
            out_hbm.at[pl.ds(base + (K + 2) * MAXC, MAXC)],
            sem,
        ),
    ]
    for cp in copies:
        cp.wait()


def kernel(cycle_emb, strength_emb, seq_len, taal_cycle_len):
    max_cycle = cycle_emb.shape[0]
    taal = jnp.asarray(taal_cycle_len, jnp.int32)
    cyc = jnp.minimum(taal, jnp.int32(max_cycle))
    j16 = jnp.arange(MAXC, dtype=jnp.int32)
    ctab = jnp.take(cycle_emb, j16 % cyc, axis=0)
    srows = jnp.take(strength_emb, jnp.where(j16 % taal == 0, 0, 3), axis=0)
    sc = pl.kernel(
        _sc_body,
        out_type=jax.ShapeDtypeStruct((SEQ, D_MODEL), jnp.float32),
        scratch_types=[
            pltpu.VMEM((1, D_MODEL), jnp.float32),
            pltpu.VMEM((1, D_MODEL), jnp.float32),
            pltpu.VMEM((KT * MAXC, D_MODEL), jnp.float32),
            pltpu.VMEM_SHARED((K * MAXC, D_MODEL), jnp.float32),
            pltpu.SemaphoreType.DMA,
        ],
        mesh=plsc.VectorSubcoreMesh(core_axis_name="c", subcore_axis_name="s"),
    )
    return sc(ctab, srows)[None, ...]


# speedup vs baseline: 1.0121x; 1.0121x over previous
import jax
import jax.numpy as jnp
from jax import lax
from jax.experimental import pallas as pl
from jax.experimental.pallas import tpu as pltpu
from jax.experimental.pallas import tpu_sc as plsc

D_MODEL = 2048
SEQ = 8192
MAXC = 16
LANES = 16
NC = 2
NS = 16
NW = NC * NS
ROWS_W = SEQ // NW
K = 4
KT = 1


def _sc_body(ctab_hbm, srows_hbm, out_hbm, row_v, srow_v, tile_v, shared, sem):
    cid = lax.axis_index("c")
    sid = lax.axis_index("s")
    wid = sid * NC + cid
    base = wid * ROWS_W
    pltpu.sync_copy(ctab_hbm.at[pl.ds(sid, 1)], row_v)
    pltpu.sync_copy(srows_hbm.at[pl.ds(sid, 1)], srow_v)

    def add_chunk(t, carry):
        sl = pl.ds(t * LANES, LANES)
        row_v[0, sl] = row_v[0, sl] + srow_v[0, sl]
        return carry

    lax.fori_loop(0, D_MODEL // LANES, add_chunk, 0)
    for k in range(K):
        pltpu.sync_copy(row_v, shared.at[pl.ds(k * MAXC + sid, 1)])
    plsc.subcore_barrier()
    copies = [
        pltpu.async_copy(shared, out_hbm.at[pl.ds(base, K * MAXC)], sem),
        pltpu.async_copy(
            shared.at[pl.ds(0, 2 * MAXC)],
            out_hbm.at[pl.ds(base + K * MAXC, 2 * MAXC)],
            sem,
        ),
        pltpu.async_copy(
            shared.at[pl.ds(0, MAXC)],
            out_hbm.at[pl.ds(base + (K + 2) * MAXC, MAXC)],
            sem,
        ),
    ]
    pltpu.sync_copy(shared.at[pl.ds(0, KT * MAXC)], tile_v)
    sp_rows = (K + 3) * MAXC
    copies += [
        pltpu.async_copy(
            tile_v,
            out_hbm.at[pl.ds(base + sp_rows + t * (KT * MAXC), KT * MAXC)],
            sem,
        )
        for t in range(9)
    ]
    for cp in copies:
        cp.wait()


def kernel(cycle_emb, strength_emb, seq_len, taal_cycle_len):
    max_cycle = cycle_emb.shape[0]
    taal = jnp.asarray(taal_cycle_len, jnp.int32)
    cyc = jnp.minimum(taal, jnp.int32(max_cycle))
    j16 = jnp.arange(MAXC, dtype=jnp.int32)
    ctab = jnp.take(cycle_emb, j16 % cyc, axis=0)
    srows = jnp.take(strength_emb, jnp.where(j16 % taal == 0, 0, 3), axis=0)
    sc = pl.kernel(
        _sc_body,
        out_type=jax.ShapeDtypeStruct((SEQ, D_MODEL), jnp.float32),
        scratch_types=[
            pltpu.VMEM((1, D_MODEL), jnp.float32),
            pltpu.VMEM((1, D_MODEL), jnp.float32),
            pltpu.VMEM((KT * MAXC, D_MODEL), jnp.float32),
            pltpu.VMEM_SHARED((K * MAXC, D_MODEL), jnp.float32),
            pltpu.SemaphoreType.DMA,
        ],
        mesh=plsc.VectorSubcoreMesh(core_axis_name="c", subcore_axis_name="s"),
    )
    return sc(ctab, srows)[None, ...]
